# baseline (speedup 1.0000x reference)
import functools

import jax
import jax.numpy as jnp
from jax import lax
from jax.experimental import pallas as pl
from jax.experimental.pallas import tpu as pltpu
from jax.experimental.pallas import tpu_sc as plsc

N = 50000
D = 256
S = 512
RA = 2000
NBA = N // RA
CH = 80
NCH = N // CH
NW = 32
WSC = 264
NSL = 3
NG = CH // 16


def _scores_body(x_ref, wt_ref, bias_ref, s_ref, st_ref, m_ref, z_ref):
    i = pl.program_id(0)

    @pl.when(i == 0)
    def _init():
        m_ref[0] = -jnp.inf
        z_ref[0] = 0.0

    xb = x_ref[...]
    srow = lax.dot_general(wt_ref[...], xb, (((1,), (1,)), ((), ())),
                           preferred_element_type=jnp.float32)
    srow = srow + bias_ref[0, 0]
    s_ref[...] = srow.reshape(1, 1, RA)
    m_old = m_ref[0]
    m_new = jnp.maximum(m_old, jnp.max(srow))
    z_ref[0] = z_ref[0] * jnp.exp(m_old - m_new) + jnp.sum(jnp.exp(srow - m_new))
    m_ref[0] = m_new

    @pl.when(i == NBA - 1)
    def _fin():
        st_ref[...] = jnp.concatenate(
            [jnp.full((1, 16), m_new, jnp.float32),
             jnp.full((1, 16), 1.0 / z_ref[0], jnp.float32)], axis=0)


def _sc_body(x_hbm, s_hbm, b_hbm, m_hbm, zi_hbm, zero_hbm,
             part_hbm, md_hbm, xv, sv, bv, mv, zv, mdv, acc):
    cid = lax.axis_index("c")
    sid = lax.axis_index("s")
    wid = cid * 16 + sid
    pltpu.sync_copy(zero_hbm, acc)
    pltpu.sync_copy(m_hbm, mv)
    pltpu.sync_copy(zi_hbm, zv)

    c0 = wid * NCH // NW
    c1 = (wid + 1) * NCH // NW

    def flush(wb, sl):
        pltpu.sync_copy(acc, part_hbm.at[pl.ds((wid * NSL + sl) * WSC, WSC)])
        mdv[...] = jnp.full((16,), jnp.clip(wb, 0, S - WSC), jnp.int32)
        pltpu.sync_copy(mdv, md_hbm.at[pl.ds((wid * NSL + sl) * 16, 16)])
        pltpu.sync_copy(zero_hbm, acc)

    def chunk(ci, carry):
        off = ci * CH
        pltpu.sync_copy(x_hbm.at[pl.ds(off, CH)], xv)
        pltpu.sync_copy(s_hbm.at[pl.ds(off, CH)], sv)
        pltpu.sync_copy(b_hbm.at[pl.ds(off, CH)], bv)
        mvv = mv[...]
        zvv = zv[...]
        for g2 in range(NG):
            sv[pl.ds(g2 * 16, 16)] = jnp.exp(sv[pl.ds(g2 * 16, 16)] - mvv) * zvv

        def group(g, carry2):
            wb, sl = carry2
            bvec = bv[pl.ds(g * 16, 16)]
            wvec = sv[pl.ds(g * 16, 16)]
            rr = g * 16
            for j in range(16):
                seg = bvec[j]
                pred = seg > wb + (WSC - 1)

                @pl.when(pred)
                def _fl():
                    flush(wb, sl)

                wb = jnp.where(pred, jnp.minimum((seg // 8) * 8, S - WSC), wb)
                sl = jnp.where(pred, sl + 1, sl)
                ofs = seg - wb
                wbc = jnp.full((16,), wvec[j])
                for k in range(D // 16):
                    cs = pl.ds(k * 16, 16)
                    acc[ofs, cs] = acc[ofs, cs] + wbc * xv[rr + j, cs]
            return (wb, sl)

        return lax.fori_loop(0, NG, group, carry)

    wb, sl = lax.fori_loop(c0, c1, chunk, (jnp.int32(-(2 ** 30)), jnp.int32(0)))
    flush(wb, sl)
    sl = sl + 1

    @pl.when(sl < NSL)
    def _cleanup():
        flush(wb, sl)


def _combine_body(md_ref, p_ref, o_ref):
    t = pl.program_id(0)

    @pl.when(t == 0)
    def _init():
        o_ref[...] = jnp.zeros_like(o_ref)

    wb = (md_ref[t, 0] // 8) * 8
    o_ref[pl.ds(wb, WSC), :] = o_ref[pl.ds(wb, WSC), :] + p_ref[0]


def kernel(x, batch, W, b):
    scores3, stats = pl.pallas_call(
        _scores_body,
        grid=(NBA,),
        in_specs=[
            pl.BlockSpec((RA, D), lambda i: (i, 0)),
            pl.BlockSpec((1, D), lambda i: (0, 0)),
            pl.BlockSpec((1, 1), lambda i: (0, 0)),
        ],
        out_specs=[
            pl.BlockSpec((1, 1, RA), lambda i: (i, 0, 0)),
            pl.BlockSpec((2, 16), lambda i: (0, 0)),
        ],
        out_shape=[
            jax.ShapeDtypeStruct((NBA, 1, RA), jnp.float32),
            jax.ShapeDtypeStruct((2, 16), jnp.float32),
        ],
        scratch_shapes=[pltpu.SMEM((1,), jnp.float32),
                        pltpu.SMEM((1,), jnp.float32)],
    )(x, W.reshape(1, D), b.reshape(1, 1))

    scores = scores3.reshape(N)
    mv = stats[0]
    zi = stats[1]
    bi = batch.astype(jnp.int32)
    zero = jnp.zeros((WSC, D), jnp.float32)

    mesh = plsc.VectorSubcoreMesh(core_axis_name="c", subcore_axis_name="s")
    sc_kernel = functools.partial(
        pl.kernel, mesh=mesh,
        out_type=[
            jax.ShapeDtypeStruct((NW * NSL * WSC, D), jnp.float32),
            jax.ShapeDtypeStruct((NW * NSL * 16,), jnp.int32),
        ],
        scratch_types=[
            pltpu.VMEM((CH, D), jnp.float32),
            pltpu.VMEM((CH,), jnp.float32),
            pltpu.VMEM((CH,), jnp.int32),
            pltpu.VMEM((16,), jnp.float32),
            pltpu.VMEM((16,), jnp.float32),
            pltpu.VMEM((16,), jnp.int32),
            pltpu.VMEM((WSC, D), jnp.float32),
        ],
    )(_sc_body)
    partials, md = sc_kernel(x, scores, bi, mv, zi, zero)

    return pl.pallas_call(
        _combine_body,
        grid=(NW * NSL,),
        in_specs=[
            pl.BlockSpec((NW * NSL, 16), lambda t: (0, 0),
                         memory_space=pltpu.SMEM),
            pl.BlockSpec((1, WSC, D), lambda t: (t, 0, 0)),
        ],
        out_specs=pl.BlockSpec((S, D), lambda t: (0, 0)),
        out_shape=jax.ShapeDtypeStruct((S, D), jnp.float32),
    )(md.reshape(NW * NSL, 16), partials.reshape(NW * NSL, WSC, D))


# speedup vs baseline: 17.7258x; 17.7258x over previous
import jax
import jax.numpy as jnp
from jax import lax
from jax.experimental import pallas as pl
from jax.experimental.pallas import tpu as pltpu

N = 50000
D = 256
S = 512
R = 5000
NB = N // R
WIN = 80


def _body(x_ref, bseg_ref, bsm_ref, wt_ref, bias_ref, out_ref, m_ref, z_ref):
    i = pl.program_id(0)

    @pl.when(i == 0)
    def _init():
        m_ref[0] = -jnp.inf
        z_ref[0] = 0.0
        out_ref[...] = jnp.zeros_like(out_ref)

    xb = x_ref[...]
    srow = lax.dot_general(wt_ref[...], xb, (((1,), (1,)), ((), ())),
                           preferred_element_type=jnp.float32)
    srow = srow + bias_ref[0, 0]

    m_old = m_ref[0]
    m_new = jnp.maximum(m_old, jnp.max(srow))
    p = jnp.exp(srow - m_new)
    z_ref[0] = z_ref[0] * jnp.exp(m_old - m_new) + jnp.sum(p)
    m_ref[0] = m_new

    @pl.when(jnp.logical_and(i > 0, m_new > m_old))
    def _rescale():
        out_ref[...] = out_ref[...] * jnp.exp(m_old - m_new)

    seg = bseg_ref[0, 0, :]
    base8 = jnp.minimum((bsm_ref[0, 0, 0] // 8) * 8, S - WIN)
    hi = bsm_ref[0, 0, R - 1]
    in_window = hi - base8 < WIN

    @pl.when(in_window)
    def _fast():
        offs = seg - base8
        wmat = jnp.where(
            lax.broadcasted_iota(jnp.int32, (WIN, R), 0) == offs[None, :],
            p, 0.0)
        contrib = jnp.dot(wmat, xb, preferred_element_type=jnp.float32)
        out_ref[pl.ds(base8, WIN), :] = out_ref[pl.ds(base8, WIN), :] + contrib

    @pl.when(jnp.logical_not(in_window))
    def _slow():
        wmat = jnp.where(
            lax.broadcasted_iota(jnp.int32, (S, R), 0) == seg[None, :],
            p, 0.0)
        contrib = jnp.dot(wmat, xb, preferred_element_type=jnp.float32)
        out_ref[...] = out_ref[...] + contrib

    @pl.when(i == NB - 1)
    def _fin():
        out_ref[...] = out_ref[...] * (1.0 / z_ref[0])


def kernel(x, batch, W, b):
    b3 = batch.astype(jnp.int32).reshape(NB, 1, R)
    return pl.pallas_call(
        _body,
        grid=(NB,),
        in_specs=[
            pl.BlockSpec((R, D), lambda i: (i, 0)),
            pl.BlockSpec((1, 1, R), lambda i: (i, 0, 0)),
            pl.BlockSpec((1, 1, R), lambda i: (i, 0, 0),
                         memory_space=pltpu.SMEM),
            pl.BlockSpec((1, D), lambda i: (0, 0)),
            pl.BlockSpec((1, 1), lambda i: (0, 0)),
        ],
        out_specs=pl.BlockSpec((S, D), lambda i: (0, 0)),
        out_shape=jax.ShapeDtypeStruct((S, D), jnp.float32),
        scratch_shapes=[pltpu.SMEM((1,), jnp.float32),
                        pltpu.SMEM((1,), jnp.float32)],
    )(x, b3, b3, W.reshape(1, D), b.reshape(1, 1))


# speedup vs baseline: 17.9796x; 1.0143x over previous
import jax
import jax.numpy as jnp
from jax import lax
from jax.experimental import pallas as pl
from jax.experimental.pallas import tpu as pltpu

N = 50000
D = 256
S = 512
R = 5000
NB = N // R
WIN = 128


def _body(x_ref, bseg_ref, bsm_ref, wt_ref, bias_ref, out_ref, m_ref, z_ref):
    i = pl.program_id(0)

    @pl.when(i == 0)
    def _init():
        m_ref[0] = -jnp.inf
        z_ref[0] = 0.0
        out_ref[...] = jnp.zeros_like(out_ref)

    xb = x_ref[...]
    srow = lax.dot_general(wt_ref[...], xb, (((1,), (1,)), ((), ())),
                           preferred_element_type=jnp.float32)
    srow = srow + bias_ref[0, 0]

    m_old = m_ref[0]
    m_new = jnp.maximum(m_old, jnp.max(srow))
    p = jnp.exp(srow - m_new)
    z_ref[0] = z_ref[0] * jnp.exp(m_old - m_new) + jnp.sum(p)
    m_ref[0] = m_new

    @pl.when(jnp.logical_and(i > 0, m_new > m_old))
    def _rescale():
        out_ref[...] = out_ref[...] * jnp.exp(m_old - m_new)

    seg = bseg_ref[0, 0, :]
    base8 = jnp.minimum((bsm_ref[0, 0, 0] // 8) * 8, S - WIN)
    hi = bsm_ref[0, 0, R - 1]
    in_window = hi - base8 < WIN

    @pl.when(in_window)
    def _fast():
        offs = seg - base8
        wmat = jnp.where(
            lax.broadcasted_iota(jnp.int32, (WIN, R), 0) == offs[None, :],
            p, 0.0)
        contrib = jnp.dot(wmat, xb, preferred_element_type=jnp.float32)
        out_ref[pl.ds(base8, WIN), :] = out_ref[pl.ds(base8, WIN), :] + contrib

    @pl.when(jnp.logical_not(in_window))
    def _slow():
        wmat = jnp.where(
            lax.broadcasted_iota(jnp.int32, (S, R), 0) == seg[None, :],
            p, 0.0)
        contrib = jnp.dot(wmat, xb, preferred_element_type=jnp.float32)
        out_ref[...] = out_ref[...] + contrib

    @pl.when(i == NB - 1)
    def _fin():
        out_ref[...] = out_ref[...] * (1.0 / z_ref[0])


def kernel(x, batch, W, b):
    b3 = batch.astype(jnp.int32).reshape(NB, 1, R)
    return pl.pallas_call(
        _body,
        grid=(NB,),
        in_specs=[
            pl.BlockSpec((R, D), lambda i: (i, 0)),
            pl.BlockSpec((1, 1, R), lambda i: (i, 0, 0)),
            pl.BlockSpec((1, 1, R), lambda i: (i, 0, 0),
                         memory_space=pltpu.SMEM),
            pl.BlockSpec((1, D), lambda i: (0, 0)),
            pl.BlockSpec((1, 1), lambda i: (0, 0)),
        ],
        out_specs=pl.BlockSpec((S, D), lambda i: (0, 0)),
        out_shape=jax.ShapeDtypeStruct((S, D), jnp.float32),
        scratch_shapes=[pltpu.SMEM((1,), jnp.float32),
                        pltpu.SMEM((1,), jnp.float32)],
    )(x, b3, b3, W.reshape(1, D), b.reshape(1, 1))
